# Initial kernel scaffold; baseline (speedup 1.0000x reference)
import jax
import jax.numpy as jnp
from jax.experimental import pallas as pl


def kernel(x, edge_index, W1, b1, g1, be1, W2, b2, g2, be2, W3, b3, g3, be3, Wc1, bc1, Wc2, bc2):
    raise NotImplementedError("write your pallas kernel here")


# speedup vs baseline: 7.6282x; 7.6282x over previous
import functools

import jax
import jax.numpy as jnp
from jax import lax
from jax.experimental import pallas as pl
from jax.experimental.pallas import tpu as pltpu
from jax.experimental.pallas import tpu_sc as plsc

NN = 10000
EE = 320000
DIN = 128
HF = 256
HH = 128
NP = 10240
SLAB = NP // 16
TRASH = NN

ECHUNK = 128
IBLK = 16
EPS = EE // 16
NOB = (EPS + IBLK * ECHUNK - 1) // (IBLK * ECHUNK)
EROWS = NOB * IBLK
EPW = EE // 32
DROWS = (EPW + ECHUNK - 1) // ECHUNK

BR = 1280
GRID = NP // BR


@functools.cache
def _sc_mesh():
    return plsc.VectorSubcoreMesh(core_axis_name="c", subcore_axis_name="s")


_SC_PARAMS = pltpu.CompilerParams(use_tc_tiling_on_sc=False)


@functools.cache
def _deg_kernel():
    @functools.partial(
        pl.kernel,
        out_type=jax.ShapeDtypeStruct((2, NP, 16), jnp.float32),
        mesh=_sc_mesh(),
        compiler_params=_SC_PARAMS,
        scratch_types=(
            pltpu.VMEM_SHARED((NP, 16), jnp.float32),
            pltpu.VMEM((DROWS, ECHUNK), jnp.int32),
            pltpu.VMEM((ECHUNK, 16), jnp.float32),
            pltpu.VMEM((SLAB, 16), jnp.float32),
            pltpu.VMEM((ECHUNK,), jnp.int32),
        ),
    )
    def deg_kernel(dst32, deg_out, deg_sh, dst_v, ones_v, z_v, idx_b):
        c = lax.axis_index("c")
        s = lax.axis_index("s")
        base = s * SLAB

        def fill_ones(i, _):
            ones_v[i, :] = jnp.full((16,), 1.0, jnp.float32)
            return 0

        lax.fori_loop(0, ECHUNK, fill_ones, 0)

        def fill_zero(i, _):
            z_v[i, :] = jnp.zeros((16,), jnp.float32)
            return 0

        lax.fori_loop(0, SLAB, fill_zero, 0)

        pltpu.sync_copy(dst32.at[c * 16 + s], dst_v)
        pltpu.sync_copy(z_v, deg_sh.at[pl.ds(base, SLAB)])
        plsc.subcore_barrier()

        def body(j, _):
            for k in range(ECHUNK // 16):
                v = dst_v[j, pl.ds(k * 16, 16)]
                v = jnp.minimum(jnp.maximum(v, 0), NP - 1)
                idx_b[pl.ds(k * 16, 16)] = v
            pltpu.sync_copy(ones_v, deg_sh.at[idx_b], add=True)
            return 0

        lax.fori_loop(0, DROWS, body, 0)
        plsc.subcore_barrier()
        pltpu.sync_copy(deg_sh.at[pl.ds(base, SLAB)], z_v)
        pltpu.sync_copy(z_v, deg_out.at[c, pl.ds(base, SLAB)])

    return deg_kernel


@functools.cache
def _msg_kernel():
    @functools.partial(
        pl.kernel,
        out_type=(
            jax.ShapeDtypeStruct((NP, HH), jnp.float32),
            jax.ShapeDtypeStruct((NP, HH), jnp.float32),
        ),
        mesh=_sc_mesh(),
        compiler_params=_SC_PARAMS,
        scratch_types=(
            pltpu.VMEM_SHARED((NP, HH), jnp.float32),
            pltpu.VMEM((IBLK, ECHUNK), jnp.int32),
            pltpu.VMEM((IBLK, ECHUNK), jnp.int32),
            pltpu.VMEM((ECHUNK, HH), jnp.float32),
            pltpu.VMEM((ECHUNK,), jnp.int32),
            pltpu.VMEM((ECHUNK,), jnp.int32),
        ),
    )
    def msg_kernel(xs0, xs1, src_t, dst_t, acc0, acc1,
                   acc_sh, src_v, dst_v, rows_v, idx_b, src_b):
        c = lax.axis_index("c")
        s = lax.axis_index("s")
        base = s * SLAB

        def run(xs_hbm, acc_out):
            def init(k, _):
                off = base + k * ECHUNK
                pltpu.sync_copy(xs_hbm.at[pl.ds(off, ECHUNK)], rows_v)
                pltpu.sync_copy(rows_v, acc_sh.at[pl.ds(off, ECHUNK)])
                return 0

            lax.fori_loop(0, SLAB // ECHUNK, init, 0)
            plsc.subcore_barrier()

            def outer(ob, _):
                pltpu.sync_copy(src_t.at[s, pl.ds(ob * IBLK, IBLK)], src_v)
                pltpu.sync_copy(dst_t.at[s, pl.ds(ob * IBLK, IBLK)], dst_v)

                def body(j, _):
                    for k in range(ECHUNK // 16):
                        v = src_v[j, pl.ds(k * 16, 16)]
                        src_b[pl.ds(k * 16, 16)] = jnp.minimum(
                            jnp.maximum(v, 0), NP - 1)
                        w = dst_v[j, pl.ds(k * 16, 16)]
                        idx_b[pl.ds(k * 16, 16)] = jnp.minimum(
                            jnp.maximum(w, 0), NP - 1)
                    pltpu.sync_copy(xs_hbm.at[src_b], rows_v)
                    pltpu.sync_copy(rows_v, acc_sh.at[idx_b], add=True)
                    return 0

                lax.fori_loop(0, IBLK, body, 0)
                return 0

            lax.fori_loop(0, NOB, outer, 0)
            plsc.subcore_barrier()

            def flush(k, _):
                off = base + k * ECHUNK
                pltpu.sync_copy(acc_sh.at[pl.ds(off, ECHUNK)], rows_v)
                pltpu.sync_copy(rows_v, acc_out.at[pl.ds(off, ECHUNK)])
                return 0

            lax.fori_loop(0, SLAB // ECHUNK, flush, 0)

        @pl.when(c == 0)
        def _():
            run(xs0, acc0)

        @pl.when(c == 1)
        def _():
            run(xs1, acc1)

    return msg_kernel


def _first_body(x_ref, w_ref, deg_ref, xs0_ref, xs1_ref, dinv_ref):
    deg = deg_ref[0, :, 0] + deg_ref[1, :, 0] + 1.0
    dinv = lax.rsqrt(deg)[:, None]
    h = jnp.dot(x_ref[...], w_ref[...], preferred_element_type=jnp.float32)
    xs = h * dinv
    xs0_ref[...] = xs[:, :HH]
    xs1_ref[...] = xs[:, HH:]
    dinv_ref[...] = jnp.broadcast_to(dinv, (BR, HF))


@functools.cache
def _first_call():
    return pl.pallas_call(
        _first_body,
        grid=(GRID,),
        in_specs=[
            pl.BlockSpec((BR, DIN), lambda i: (i, 0)),
            pl.BlockSpec((DIN, HF), lambda i: (0, 0)),
            pl.BlockSpec((2, BR, 16), lambda i: (0, i, 0)),
        ],
        out_specs=[
            pl.BlockSpec((BR, HH), lambda i: (i, 0)),
            pl.BlockSpec((BR, HH), lambda i: (i, 0)),
            pl.BlockSpec((BR, HF), lambda i: (i, 0)),
        ],
        out_shape=[
            jax.ShapeDtypeStruct((NP, HH), jnp.float32),
            jax.ShapeDtypeStruct((NP, HH), jnp.float32),
            jax.ShapeDtypeStruct((NP, HF), jnp.float32),
        ],
    )


def _norm_relu(acc0, acc1, dinv, b, g, be):
    h = jnp.concatenate([acc0, acc1], axis=1) * dinv + b
    mu = jnp.mean(h, axis=1, keepdims=True)
    var = jnp.mean((h - mu) * (h - mu), axis=1, keepdims=True)
    y = (h - mu) * lax.rsqrt(var + 1e-5) * g + be
    return jnp.maximum(y, 0.0)


def _mid_body(acc0_ref, acc1_ref, dinv_ref, b_ref, g_ref, be_ref, w_ref,
              xs0_ref, xs1_ref):
    dinv = dinv_ref[...]
    r = _norm_relu(acc0_ref[...], acc1_ref[...], dinv,
                   b_ref[...], g_ref[...], be_ref[...])
    o = jnp.dot(r, w_ref[...], preferred_element_type=jnp.float32) * dinv
    xs0_ref[...] = o[:, :HH]
    xs1_ref[...] = o[:, HH:]


@functools.cache
def _mid_call():
    vec = pl.BlockSpec((1, HF), lambda i: (0, 0))
    return pl.pallas_call(
        _mid_body,
        grid=(GRID,),
        in_specs=[
            pl.BlockSpec((BR, HH), lambda i: (i, 0)),
            pl.BlockSpec((BR, HH), lambda i: (i, 0)),
            pl.BlockSpec((BR, HF), lambda i: (i, 0)),
            vec, vec, vec,
            pl.BlockSpec((HF, HF), lambda i: (0, 0)),
        ],
        out_specs=[
            pl.BlockSpec((BR, HH), lambda i: (i, 0)),
            pl.BlockSpec((BR, HH), lambda i: (i, 0)),
        ],
        out_shape=[
            jax.ShapeDtypeStruct((NP, HH), jnp.float32),
            jax.ShapeDtypeStruct((NP, HH), jnp.float32),
        ],
    )


def _last_body(acc0_ref, acc1_ref, dinv_ref, b_ref, g_ref, be_ref,
               wc1_ref, bc1_ref, wc2_ref, bc2_ref, out_ref):
    r = _norm_relu(acc0_ref[...], acc1_ref[...], dinv_ref[...],
                   b_ref[...], g_ref[...], be_ref[...])
    m = jnp.maximum(
        jnp.dot(r, wc1_ref[...], preferred_element_type=jnp.float32)
        + bc1_ref[...], 0.0)
    out_ref[...] = (
        jnp.dot(m, wc2_ref[...], preferred_element_type=jnp.float32)
        + bc2_ref[...])


@functools.cache
def _last_call():
    vec = pl.BlockSpec((1, HF), lambda i: (0, 0))
    return pl.pallas_call(
        _last_body,
        grid=(GRID,),
        in_specs=[
            pl.BlockSpec((BR, HH), lambda i: (i, 0)),
            pl.BlockSpec((BR, HH), lambda i: (i, 0)),
            pl.BlockSpec((BR, HF), lambda i: (i, 0)),
            vec, vec, vec,
            pl.BlockSpec((HF, HF), lambda i: (0, 0)),
            vec,
            pl.BlockSpec((HF, 1), lambda i: (0, 0)),
            pl.BlockSpec((1, 1), lambda i: (0, 0)),
        ],
        out_specs=[pl.BlockSpec((BR, 1), lambda i: (i, 0))],
        out_shape=[jax.ShapeDtypeStruct((NP, 1), jnp.float32)],
    )


def _pad_edges(idx, nway, rows, fill):
    per = EE // nway
    a = idx.reshape(nway, per)
    pad = rows * ECHUNK - per
    a = jnp.pad(a, ((0, 0), (0, pad)), constant_values=fill)
    return a.reshape(nway, rows, ECHUNK)


def kernel(x, edge_index, W1, b1, g1, be1, W2, b2, g2, be2, W3, b3, g3, be3,
           Wc1, bc1, Wc2, bc2):
    src = edge_index[0]
    dst = edge_index[1]
    src16 = _pad_edges(src, 16, EROWS, 0)
    dst16 = _pad_edges(dst, 16, EROWS, TRASH)
    dst32 = _pad_edges(dst, 32, DROWS, TRASH)
    xpad = jnp.pad(x, ((0, NP - NN), (0, 0)))

    deg = _deg_kernel()(dst32)
    xs0, xs1, dinv = _first_call()(xpad, W1, deg)

    msg = _msg_kernel()
    mid = _mid_call()
    acc0, acc1 = msg(xs0, xs1, src16, dst16)
    xs0, xs1 = mid(acc0, acc1, dinv, b1.reshape(1, HF), g1.reshape(1, HF),
                   be1.reshape(1, HF), W2)
    acc0, acc1 = msg(xs0, xs1, src16, dst16)
    xs0, xs1 = mid(acc0, acc1, dinv, b2.reshape(1, HF), g2.reshape(1, HF),
                   be2.reshape(1, HF), W3)
    acc0, acc1 = msg(xs0, xs1, src16, dst16)
    (logits,) = _last_call()(acc0, acc1, dinv, b3.reshape(1, HF),
                             g3.reshape(1, HF), be3.reshape(1, HF),
                             Wc1, bc1.reshape(1, HF), Wc2,
                             bc2.reshape(1, 1))
    return logits[:NN]


# speedup vs baseline: 9.4208x; 1.2350x over previous
import functools

import jax
import jax.numpy as jnp
from jax import lax
from jax.experimental import pallas as pl
from jax.experimental.pallas import tpu as pltpu
from jax.experimental.pallas import tpu_sc as plsc

NN = 10000
EE = 320000
DIN = 128
HF = 256
HH = 128
NP = 10240
SLAB = NP // 16
TRASH = NN

ECHUNK = 128
IBLK = 16
EPS = EE // 16
NOB = (EPS + IBLK * ECHUNK - 1) // (IBLK * ECHUNK)
EROWS = NOB * IBLK
EPW = EE // 32
DROWS = (EPW + ECHUNK - 1) // ECHUNK

BR = 1280
GRID = NP // BR


@functools.cache
def _sc_mesh():
    return plsc.VectorSubcoreMesh(core_axis_name="c", subcore_axis_name="s")


_SC_PARAMS = pltpu.CompilerParams(use_tc_tiling_on_sc=False)


@functools.cache
def _deg_kernel():
    @functools.partial(
        pl.kernel,
        out_type=jax.ShapeDtypeStruct((2, NP, 16), jnp.float32),
        mesh=_sc_mesh(),
        compiler_params=_SC_PARAMS,
        scratch_types=(
            pltpu.VMEM_SHARED((NP, 16), jnp.float32),
            pltpu.VMEM((DROWS, ECHUNK), jnp.int32),
            pltpu.VMEM((ECHUNK, 16), jnp.float32),
            pltpu.VMEM((SLAB, 16), jnp.float32),
            pltpu.VMEM((ECHUNK,), jnp.int32),
        ),
    )
    def deg_kernel(dst32, deg_out, deg_sh, dst_v, ones_v, z_v, idx_b):
        c = lax.axis_index("c")
        s = lax.axis_index("s")
        base = s * SLAB

        def fill_ones(i, _):
            ones_v[i, :] = jnp.full((16,), 1.0, jnp.float32)
            return 0

        lax.fori_loop(0, ECHUNK, fill_ones, 0)

        def fill_zero(i, _):
            z_v[i, :] = jnp.zeros((16,), jnp.float32)
            return 0

        lax.fori_loop(0, SLAB, fill_zero, 0)

        pltpu.sync_copy(dst32.at[c * 16 + s], dst_v)
        pltpu.sync_copy(z_v, deg_sh.at[pl.ds(base, SLAB)])
        plsc.subcore_barrier()

        def body(j, _):
            for k in range(ECHUNK // 16):
                v = dst_v[j, pl.ds(k * 16, 16)]
                v = jnp.minimum(jnp.maximum(v, 0), NP - 1)
                idx_b[pl.ds(k * 16, 16)] = v
            pltpu.sync_copy(ones_v, deg_sh.at[idx_b], add=True)
            return 0

        lax.fori_loop(0, DROWS, body, 0)
        plsc.subcore_barrier()
        pltpu.sync_copy(deg_sh.at[pl.ds(base, SLAB)], z_v)
        pltpu.sync_copy(z_v, deg_out.at[c, pl.ds(base, SLAB)])

    return deg_kernel


@functools.cache
def _msg_kernel():
    @functools.partial(
        pl.kernel,
        out_type=(
            jax.ShapeDtypeStruct((NP, HH), jnp.float32),
            jax.ShapeDtypeStruct((NP, HH), jnp.float32),
        ),
        mesh=_sc_mesh(),
        compiler_params=_SC_PARAMS,
        scratch_types=(
            pltpu.VMEM_SHARED((NP, HH), jnp.float32),
            pltpu.VMEM((IBLK, ECHUNK), jnp.int32),
            pltpu.VMEM((IBLK, ECHUNK), jnp.int32),
            pltpu.VMEM((ECHUNK, HH), jnp.float32),
            pltpu.VMEM((ECHUNK, HH), jnp.float32),
            pltpu.VMEM((ECHUNK,), jnp.int32),
            pltpu.VMEM((ECHUNK,), jnp.int32),
            pltpu.VMEM((ECHUNK,), jnp.int32),
            pltpu.VMEM((ECHUNK,), jnp.int32),
            pltpu.SemaphoreType.DMA,
            pltpu.SemaphoreType.DMA,
            pltpu.SemaphoreType.DMA,
            pltpu.SemaphoreType.DMA,
        ),
    )
    def msg_kernel(xs0, xs1, src_t, dst_t, acc0, acc1,
                   acc_sh, src_v, dst_v, rows_v0, rows_v1,
                   idx_b0, idx_b1, src_b0, src_b1,
                   gsem0, gsem1, ssem0, ssem1):
        c = lax.axis_index("c")
        s = lax.axis_index("s")
        base = s * SLAB
        rows = (rows_v0, rows_v1)
        idx_b = (idx_b0, idx_b1)
        src_b = (src_b0, src_b1)
        gsem = (gsem0, gsem1)
        ssem = (ssem0, ssem1)

        def run(xs_hbm, acc_out):
            def init(k, _):
                off = base + k * ECHUNK
                pltpu.sync_copy(xs_hbm.at[pl.ds(off, ECHUNK)], rows_v0)
                pltpu.sync_copy(rows_v0, acc_sh.at[pl.ds(off, ECHUNK)])
                return 0

            lax.fori_loop(0, SLAB // ECHUNK, init, 0)
            plsc.subcore_barrier()

            def fill(j, p):
                for k in range(ECHUNK // 16):
                    v = src_v[j, pl.ds(k * 16, 16)]
                    src_b[p][pl.ds(k * 16, 16)] = jnp.minimum(
                        jnp.maximum(v, 0), NP - 1)
                    w = dst_v[j, pl.ds(k * 16, 16)]
                    idx_b[p][pl.ds(k * 16, 16)] = jnp.minimum(
                        jnp.maximum(w, 0), NP - 1)

            def outer(ob, _):
                pltpu.sync_copy(src_t.at[s, pl.ds(ob * IBLK, IBLK)], src_v)
                pltpu.sync_copy(dst_t.at[s, pl.ds(ob * IBLK, IBLK)], dst_v)
                fill(0, 0)
                g = [None, None]
                sc = [None, None]
                g[0] = pltpu.async_copy(xs_hbm.at[src_b[0]], rows[0], gsem[0])
                for j in range(IBLK):
                    p = j & 1
                    q = 1 - p
                    if j + 1 < IBLK:
                        if sc[q] is not None:
                            sc[q].wait()
                            sc[q] = None
                        fill(j + 1, q)
                        g[q] = pltpu.async_copy(
                            xs_hbm.at[src_b[q]], rows[q], gsem[q])
                    g[p].wait()
                    sc[p] = pltpu.async_copy(
                        rows[p], acc_sh.at[idx_b[p]], ssem[p], add=True)
                sc[0].wait()
                sc[1].wait()
                return 0

            lax.fori_loop(0, NOB, outer, 0)
            plsc.subcore_barrier()

            def flush(k, _):
                off = base + k * ECHUNK
                pltpu.sync_copy(acc_sh.at[pl.ds(off, ECHUNK)], rows_v0)
                pltpu.sync_copy(rows_v0, acc_out.at[pl.ds(off, ECHUNK)])
                return 0

            lax.fori_loop(0, SLAB // ECHUNK, flush, 0)

        @pl.when(c == 0)
        def _():
            run(xs0, acc0)

        @pl.when(c == 1)
        def _():
            run(xs1, acc1)

    return msg_kernel


def _first_body(x_ref, w_ref, deg_ref, xs0_ref, xs1_ref, dinv_ref):
    deg = deg_ref[0, :, 0] + deg_ref[1, :, 0] + 1.0
    dinv = lax.rsqrt(deg)[:, None]
    h = jnp.dot(x_ref[...], w_ref[...], preferred_element_type=jnp.float32)
    xs = h * dinv
    xs0_ref[...] = xs[:, :HH]
    xs1_ref[...] = xs[:, HH:]
    dinv_ref[...] = jnp.broadcast_to(dinv, (BR, HF))


@functools.cache
def _first_call():
    return pl.pallas_call(
        _first_body,
        grid=(GRID,),
        in_specs=[
            pl.BlockSpec((BR, DIN), lambda i: (i, 0)),
            pl.BlockSpec((DIN, HF), lambda i: (0, 0)),
            pl.BlockSpec((2, BR, 16), lambda i: (0, i, 0)),
        ],
        out_specs=[
            pl.BlockSpec((BR, HH), lambda i: (i, 0)),
            pl.BlockSpec((BR, HH), lambda i: (i, 0)),
            pl.BlockSpec((BR, HF), lambda i: (i, 0)),
        ],
        out_shape=[
            jax.ShapeDtypeStruct((NP, HH), jnp.float32),
            jax.ShapeDtypeStruct((NP, HH), jnp.float32),
            jax.ShapeDtypeStruct((NP, HF), jnp.float32),
        ],
    )


def _norm_relu(acc0, acc1, dinv, b, g, be):
    h = jnp.concatenate([acc0, acc1], axis=1) * dinv + b
    mu = jnp.mean(h, axis=1, keepdims=True)
    var = jnp.mean((h - mu) * (h - mu), axis=1, keepdims=True)
    y = (h - mu) * lax.rsqrt(var + 1e-5) * g + be
    return jnp.maximum(y, 0.0)


def _mid_body(acc0_ref, acc1_ref, dinv_ref, b_ref, g_ref, be_ref, w_ref,
              xs0_ref, xs1_ref):
    dinv = dinv_ref[...]
    r = _norm_relu(acc0_ref[...], acc1_ref[...], dinv,
                   b_ref[...], g_ref[...], be_ref[...])
    o = jnp.dot(r, w_ref[...], preferred_element_type=jnp.float32) * dinv
    xs0_ref[...] = o[:, :HH]
    xs1_ref[...] = o[:, HH:]


@functools.cache
def _mid_call():
    vec = pl.BlockSpec((1, HF), lambda i: (0, 0))
    return pl.pallas_call(
        _mid_body,
        grid=(GRID,),
        in_specs=[
            pl.BlockSpec((BR, HH), lambda i: (i, 0)),
            pl.BlockSpec((BR, HH), lambda i: (i, 0)),
            pl.BlockSpec((BR, HF), lambda i: (i, 0)),
            vec, vec, vec,
            pl.BlockSpec((HF, HF), lambda i: (0, 0)),
        ],
        out_specs=[
            pl.BlockSpec((BR, HH), lambda i: (i, 0)),
            pl.BlockSpec((BR, HH), lambda i: (i, 0)),
        ],
        out_shape=[
            jax.ShapeDtypeStruct((NP, HH), jnp.float32),
            jax.ShapeDtypeStruct((NP, HH), jnp.float32),
        ],
    )


def _last_body(acc0_ref, acc1_ref, dinv_ref, b_ref, g_ref, be_ref,
               wc1_ref, bc1_ref, wc2_ref, bc2_ref, out_ref):
    r = _norm_relu(acc0_ref[...], acc1_ref[...], dinv_ref[...],
                   b_ref[...], g_ref[...], be_ref[...])
    m = jnp.maximum(
        jnp.dot(r, wc1_ref[...], preferred_element_type=jnp.float32)
        + bc1_ref[...], 0.0)
    out_ref[...] = (
        jnp.dot(m, wc2_ref[...], preferred_element_type=jnp.float32)
        + bc2_ref[...])


@functools.cache
def _last_call():
    vec = pl.BlockSpec((1, HF), lambda i: (0, 0))
    return pl.pallas_call(
        _last_body,
        grid=(GRID,),
        in_specs=[
            pl.BlockSpec((BR, HH), lambda i: (i, 0)),
            pl.BlockSpec((BR, HH), lambda i: (i, 0)),
            pl.BlockSpec((BR, HF), lambda i: (i, 0)),
            vec, vec, vec,
            pl.BlockSpec((HF, HF), lambda i: (0, 0)),
            vec,
            pl.BlockSpec((HF, 1), lambda i: (0, 0)),
            pl.BlockSpec((1, 1), lambda i: (0, 0)),
        ],
        out_specs=[pl.BlockSpec((BR, 1), lambda i: (i, 0))],
        out_shape=[jax.ShapeDtypeStruct((NP, 1), jnp.float32)],
    )


def _pad_edges(idx, nway, rows, fill):
    per = EE // nway
    a = idx.reshape(nway, per)
    pad = rows * ECHUNK - per
    a = jnp.pad(a, ((0, 0), (0, pad)), constant_values=fill)
    return a.reshape(nway, rows, ECHUNK)


def kernel(x, edge_index, W1, b1, g1, be1, W2, b2, g2, be2, W3, b3, g3, be3,
           Wc1, bc1, Wc2, bc2):
    src = edge_index[0]
    dst = edge_index[1]
    src16 = _pad_edges(src, 16, EROWS, 0)
    dst16 = _pad_edges(dst, 16, EROWS, TRASH)
    dst32 = _pad_edges(dst, 32, DROWS, TRASH)
    xpad = jnp.pad(x, ((0, NP - NN), (0, 0)))

    deg = _deg_kernel()(dst32)
    xs0, xs1, dinv = _first_call()(xpad, W1, deg)

    msg = _msg_kernel()
    mid = _mid_call()
    acc0, acc1 = msg(xs0, xs1, src16, dst16)
    xs0, xs1 = mid(acc0, acc1, dinv, b1.reshape(1, HF), g1.reshape(1, HF),
                   be1.reshape(1, HF), W2)
    acc0, acc1 = msg(xs0, xs1, src16, dst16)
    xs0, xs1 = mid(acc0, acc1, dinv, b2.reshape(1, HF), g2.reshape(1, HF),
                   be2.reshape(1, HF), W3)
    acc0, acc1 = msg(xs0, xs1, src16, dst16)
    (logits,) = _last_call()(acc0, acc1, dinv, b3.reshape(1, HF),
                             g3.reshape(1, HF), be3.reshape(1, HF),
                             Wc1, bc1.reshape(1, HF), Wc2,
                             bc2.reshape(1, 1))
    return logits[:NN]


# speedup vs baseline: 20.0245x; 2.1256x over previous
import functools

import jax
import jax.numpy as jnp
from jax import lax
from jax.experimental import pallas as pl
from jax.experimental.pallas import tpu as pltpu
from jax.experimental.pallas import tpu_sc as plsc

NN = 10000
EE = 320000
DIN = 128
HF = 256
HH = 128
NP = 10240
SLAB = NP // 16
TRASH = NN

ECHUNK = 80
NBUF = 4
LOOKAHEAD = 3
IBLK = 25
EPS = EE // 16
NOB = EPS // (IBLK * ECHUNK)
EROWS = NOB * IBLK
DCHUNK = 128
EPW = EE // 32
DROWS = (EPW + DCHUNK - 1) // DCHUNK

BR = 1280
GRID = NP // BR


@functools.cache
def _sc_mesh():
    return plsc.VectorSubcoreMesh(core_axis_name="c", subcore_axis_name="s")


_SC_PARAMS = pltpu.CompilerParams(use_tc_tiling_on_sc=False)


@functools.cache
def _deg_kernel():
    @functools.partial(
        pl.kernel,
        out_type=jax.ShapeDtypeStruct((2, NP, 16), jnp.float32),
        mesh=_sc_mesh(),
        compiler_params=_SC_PARAMS,
        scratch_types=(
            pltpu.VMEM_SHARED((NP, 16), jnp.float32),
            pltpu.VMEM((DROWS, DCHUNK), jnp.int32),
            pltpu.VMEM((DCHUNK, 16), jnp.float32),
            pltpu.VMEM((SLAB, 16), jnp.float32),
            pltpu.VMEM((DCHUNK,), jnp.int32),
        ),
    )
    def deg_kernel(dst32, deg_out, deg_sh, dst_v, ones_v, z_v, idx_b):
        c = lax.axis_index("c")
        s = lax.axis_index("s")
        base = s * SLAB

        def fill_ones(i, _):
            ones_v[i, :] = jnp.full((16,), 1.0, jnp.float32)
            return 0

        lax.fori_loop(0, DCHUNK, fill_ones, 0)

        def fill_zero(i, _):
            z_v[i, :] = jnp.zeros((16,), jnp.float32)
            return 0

        lax.fori_loop(0, SLAB, fill_zero, 0)

        pltpu.sync_copy(dst32.at[c * 16 + s], dst_v)
        pltpu.sync_copy(z_v, deg_sh.at[pl.ds(base, SLAB)])
        plsc.subcore_barrier()

        def body(j, _):
            for k in range(DCHUNK // 16):
                v = dst_v[j, pl.ds(k * 16, 16)]
                v = jnp.minimum(jnp.maximum(v, 0), NP - 1)
                idx_b[pl.ds(k * 16, 16)] = v
            pltpu.sync_copy(ones_v, deg_sh.at[idx_b], add=True)
            return 0

        lax.fori_loop(0, DROWS, body, 0)
        plsc.subcore_barrier()
        pltpu.sync_copy(deg_sh.at[pl.ds(base, SLAB)], z_v)
        pltpu.sync_copy(z_v, deg_out.at[c, pl.ds(base, SLAB)])

    return deg_kernel


@functools.cache
def _msg_kernel():
    @functools.partial(
        pl.kernel,
        out_type=(
            jax.ShapeDtypeStruct((NP, HH), jnp.float32),
            jax.ShapeDtypeStruct((NP, HH), jnp.float32),
        ),
        mesh=_sc_mesh(),
        compiler_params=_SC_PARAMS,
        scratch_types=(
            pltpu.VMEM_SHARED((NP, HH), jnp.float32),
            pltpu.VMEM((IBLK, ECHUNK), jnp.int32),
            pltpu.VMEM((IBLK, ECHUNK), jnp.int32),
        )
        + tuple(pltpu.VMEM((ECHUNK, HH), jnp.float32) for _ in range(NBUF))
        + tuple(pltpu.VMEM((ECHUNK,), jnp.int32) for _ in range(2 * NBUF))
        + tuple(pltpu.SemaphoreType.DMA for _ in range(2 * NBUF)),
    )
    def msg_kernel(xs0, xs1, src_t, dst_t, acc0, acc1,
                   acc_sh, src_v, dst_v, *bufs):
        rows = bufs[:NBUF]
        idx_b = bufs[NBUF:2 * NBUF]
        src_b = bufs[2 * NBUF:3 * NBUF]
        gsem = bufs[3 * NBUF:4 * NBUF]
        ssem = bufs[4 * NBUF:5 * NBUF]
        c = lax.axis_index("c")
        s = lax.axis_index("s")
        base = s * SLAB
        rows_v0 = rows[0]

        def run(xs_hbm, acc_out):
            def init(k, _):
                off = base + k * ECHUNK
                pltpu.sync_copy(xs_hbm.at[pl.ds(off, ECHUNK)], rows_v0)
                pltpu.sync_copy(rows_v0, acc_sh.at[pl.ds(off, ECHUNK)])
                return 0

            lax.fori_loop(0, SLAB // ECHUNK, init, 0)
            plsc.subcore_barrier()

            def fill(j, p):
                for k in range(ECHUNK // 16):
                    v = src_v[j, pl.ds(k * 16, 16)]
                    src_b[p][pl.ds(k * 16, 16)] = jnp.minimum(
                        jnp.maximum(v, 0), NP - 1)
                    w = dst_v[j, pl.ds(k * 16, 16)]
                    idx_b[p][pl.ds(k * 16, 16)] = jnp.minimum(
                        jnp.maximum(w, 0), NP - 1)

            def outer(ob, _):
                pltpu.sync_copy(src_t.at[s, pl.ds(ob * IBLK, IBLK)], src_v)
                pltpu.sync_copy(dst_t.at[s, pl.ds(ob * IBLK, IBLK)], dst_v)
                g = [None] * NBUF
                sc = [None] * NBUF
                for t in range(LOOKAHEAD):
                    fill(t, t)
                    g[t] = pltpu.async_copy(
                        xs_hbm.at[src_b[t]], rows[t], gsem[t])
                for j in range(IBLK):
                    b = j % NBUF
                    g[b].wait()
                    sc[b] = pltpu.async_copy(
                        rows[b], acc_sh.at[idx_b[b]], ssem[b], add=True)
                    t = j + LOOKAHEAD
                    if t < IBLK:
                        tb = t % NBUF
                        if sc[tb] is not None:
                            sc[tb].wait()
                            sc[tb] = None
                        fill(t, tb)
                        g[tb] = pltpu.async_copy(
                            xs_hbm.at[src_b[tb]], rows[tb], gsem[tb])
                for b in range(NBUF):
                    if sc[b] is not None:
                        sc[b].wait()
                return 0

            lax.fori_loop(0, NOB, outer, 0)
            plsc.subcore_barrier()

            def flush(k, _):
                off = base + k * ECHUNK
                pltpu.sync_copy(acc_sh.at[pl.ds(off, ECHUNK)], rows_v0)
                pltpu.sync_copy(rows_v0, acc_out.at[pl.ds(off, ECHUNK)])
                return 0

            lax.fori_loop(0, SLAB // ECHUNK, flush, 0)

        @pl.when(c == 0)
        def _():
            run(xs0, acc0)

        @pl.when(c == 1)
        def _():
            run(xs1, acc1)

    return msg_kernel


def _first_body(x_ref, w_ref, deg_ref, xs0_ref, xs1_ref, dinv_ref):
    deg = deg_ref[0, :, 0] + deg_ref[1, :, 0] + 1.0
    dinv = lax.rsqrt(deg)[:, None]
    h = jnp.dot(x_ref[...], w_ref[...], preferred_element_type=jnp.float32)
    xs = h * dinv
    xs0_ref[...] = xs[:, :HH]
    xs1_ref[...] = xs[:, HH:]
    dinv_ref[...] = jnp.broadcast_to(dinv, (BR, HF))


@functools.cache
def _first_call():
    return pl.pallas_call(
        _first_body,
        grid=(GRID,),
        in_specs=[
            pl.BlockSpec((BR, DIN), lambda i: (i, 0)),
            pl.BlockSpec((DIN, HF), lambda i: (0, 0)),
            pl.BlockSpec((2, BR, 16), lambda i: (0, i, 0)),
        ],
        out_specs=[
            pl.BlockSpec((BR, HH), lambda i: (i, 0)),
            pl.BlockSpec((BR, HH), lambda i: (i, 0)),
            pl.BlockSpec((BR, HF), lambda i: (i, 0)),
        ],
        out_shape=[
            jax.ShapeDtypeStruct((NP, HH), jnp.float32),
            jax.ShapeDtypeStruct((NP, HH), jnp.float32),
            jax.ShapeDtypeStruct((NP, HF), jnp.float32),
        ],
    )


def _norm_relu(acc0, acc1, dinv, b, g, be):
    h = jnp.concatenate([acc0, acc1], axis=1) * dinv + b
    mu = jnp.mean(h, axis=1, keepdims=True)
    var = jnp.mean((h - mu) * (h - mu), axis=1, keepdims=True)
    y = (h - mu) * lax.rsqrt(var + 1e-5) * g + be
    return jnp.maximum(y, 0.0)


def _mid_body(acc0_ref, acc1_ref, dinv_ref, b_ref, g_ref, be_ref, w_ref,
              xs0_ref, xs1_ref):
    dinv = dinv_ref[...]
    r = _norm_relu(acc0_ref[...], acc1_ref[...], dinv,
                   b_ref[...], g_ref[...], be_ref[...])
    o = jnp.dot(r, w_ref[...], preferred_element_type=jnp.float32) * dinv
    xs0_ref[...] = o[:, :HH]
    xs1_ref[...] = o[:, HH:]


@functools.cache
def _mid_call():
    vec = pl.BlockSpec((1, HF), lambda i: (0, 0))
    return pl.pallas_call(
        _mid_body,
        grid=(GRID,),
        in_specs=[
            pl.BlockSpec((BR, HH), lambda i: (i, 0)),
            pl.BlockSpec((BR, HH), lambda i: (i, 0)),
            pl.BlockSpec((BR, HF), lambda i: (i, 0)),
            vec, vec, vec,
            pl.BlockSpec((HF, HF), lambda i: (0, 0)),
        ],
        out_specs=[
            pl.BlockSpec((BR, HH), lambda i: (i, 0)),
            pl.BlockSpec((BR, HH), lambda i: (i, 0)),
        ],
        out_shape=[
            jax.ShapeDtypeStruct((NP, HH), jnp.float32),
            jax.ShapeDtypeStruct((NP, HH), jnp.float32),
        ],
    )


def _last_body(acc0_ref, acc1_ref, dinv_ref, b_ref, g_ref, be_ref,
               wc1_ref, bc1_ref, wc2_ref, bc2_ref, out_ref):
    r = _norm_relu(acc0_ref[...], acc1_ref[...], dinv_ref[...],
                   b_ref[...], g_ref[...], be_ref[...])
    m = jnp.maximum(
        jnp.dot(r, wc1_ref[...], preferred_element_type=jnp.float32)
        + bc1_ref[...], 0.0)
    out_ref[...] = (
        jnp.dot(m, wc2_ref[...], preferred_element_type=jnp.float32)
        + bc2_ref[...])


@functools.cache
def _last_call():
    vec = pl.BlockSpec((1, HF), lambda i: (0, 0))
    return pl.pallas_call(
        _last_body,
        grid=(GRID,),
        in_specs=[
            pl.BlockSpec((BR, HH), lambda i: (i, 0)),
            pl.BlockSpec((BR, HH), lambda i: (i, 0)),
            pl.BlockSpec((BR, HF), lambda i: (i, 0)),
            vec, vec, vec,
            pl.BlockSpec((HF, HF), lambda i: (0, 0)),
            vec,
            pl.BlockSpec((HF, 1), lambda i: (0, 0)),
            pl.BlockSpec((1, 1), lambda i: (0, 0)),
        ],
        out_specs=[pl.BlockSpec((BR, 1), lambda i: (i, 0))],
        out_shape=[jax.ShapeDtypeStruct((NP, 1), jnp.float32)],
    )


def _pad_edges(idx, nway, rows, chunk, fill):
    per = EE // nway
    a = idx.reshape(nway, per)
    pad = rows * chunk - per
    a = jnp.pad(a, ((0, 0), (0, pad)), constant_values=fill)
    return a.reshape(nway, rows, chunk)


def kernel(x, edge_index, W1, b1, g1, be1, W2, b2, g2, be2, W3, b3, g3, be3,
           Wc1, bc1, Wc2, bc2):
    src = edge_index[0]
    dst = edge_index[1]
    src16 = _pad_edges(src, 16, EROWS, ECHUNK, 0)
    dst16 = _pad_edges(dst, 16, EROWS, ECHUNK, TRASH)
    dst32 = _pad_edges(dst, 32, DROWS, DCHUNK, TRASH)
    xpad = jnp.pad(x, ((0, NP - NN), (0, 0)))

    deg = _deg_kernel()(dst32)
    xs0, xs1, dinv = _first_call()(xpad, W1, deg)

    msg = _msg_kernel()
    mid = _mid_call()
    acc0, acc1 = msg(xs0, xs1, src16, dst16)
    xs0, xs1 = mid(acc0, acc1, dinv, b1.reshape(1, HF), g1.reshape(1, HF),
                   be1.reshape(1, HF), W2)
    acc0, acc1 = msg(xs0, xs1, src16, dst16)
    xs0, xs1 = mid(acc0, acc1, dinv, b2.reshape(1, HF), g2.reshape(1, HF),
                   be2.reshape(1, HF), W3)
    acc0, acc1 = msg(xs0, xs1, src16, dst16)
    (logits,) = _last_call()(acc0, acc1, dinv, b3.reshape(1, HF),
                             g3.reshape(1, HF), be3.reshape(1, HF),
                             Wc1, bc1.reshape(1, HF), Wc2,
                             bc2.reshape(1, 1))
    return logits[:NN]


# speedup vs baseline: 20.5085x; 1.0242x over previous
import functools

import jax
import jax.numpy as jnp
from jax import lax
from jax.experimental import pallas as pl
from jax.experimental.pallas import tpu as pltpu
from jax.experimental.pallas import tpu_sc as plsc

NN = 10000
EE = 320000
DIN = 128
HF = 256
HH = 128
NP = 10240
SLAB = NP // 16
TRASH = NN

ECHUNK = 80
NBUF = 4
LOOKAHEAD = 3
IBLK = 25
EPS = EE // 16
NOB = EPS // (IBLK * ECHUNK)
EROWS = NOB * IBLK
DCHUNK = 128
EPW = EE // 32
DROWS = (EPW + DCHUNK - 1) // DCHUNK

BR = 1280
GRID = NP // BR


@functools.cache
def _sc_mesh():
    return plsc.VectorSubcoreMesh(core_axis_name="c", subcore_axis_name="s")


_SC_PARAMS = pltpu.CompilerParams(use_tc_tiling_on_sc=False)


@functools.cache
def _deg_kernel():
    @functools.partial(
        pl.kernel,
        out_type=jax.ShapeDtypeStruct((2, NP, 16), jnp.float32),
        mesh=_sc_mesh(),
        compiler_params=_SC_PARAMS,
        scratch_types=(
            pltpu.VMEM_SHARED((NP, 16), jnp.float32),
            pltpu.VMEM((DROWS, DCHUNK), jnp.int32),
            pltpu.VMEM((DCHUNK, 16), jnp.float32),
            pltpu.VMEM((SLAB, 16), jnp.float32),
            pltpu.VMEM((DCHUNK,), jnp.int32),
        ),
    )
    def deg_kernel(dst32, deg_out, deg_sh, dst_v, ones_v, z_v, idx_b):
        c = lax.axis_index("c")
        s = lax.axis_index("s")
        base = s * SLAB

        def fill_ones(i, _):
            ones_v[i, :] = jnp.full((16,), 1.0, jnp.float32)
            return 0

        lax.fori_loop(0, DCHUNK, fill_ones, 0)

        def fill_zero(i, _):
            z_v[i, :] = jnp.zeros((16,), jnp.float32)
            return 0

        lax.fori_loop(0, SLAB, fill_zero, 0)

        pltpu.sync_copy(dst32.at[c * 16 + s], dst_v)
        pltpu.sync_copy(z_v, deg_sh.at[pl.ds(base, SLAB)])
        plsc.subcore_barrier()

        def body(j, _):
            for k in range(DCHUNK // 16):
                v = dst_v[j, pl.ds(k * 16, 16)]
                v = jnp.minimum(jnp.maximum(v, 0), NP - 1)
                idx_b[pl.ds(k * 16, 16)] = v
            pltpu.sync_copy(ones_v, deg_sh.at[idx_b], add=True)
            return 0

        lax.fori_loop(0, DROWS, body, 0)
        plsc.subcore_barrier()
        pltpu.sync_copy(deg_sh.at[pl.ds(base, SLAB)], z_v)
        pltpu.sync_copy(z_v, deg_out.at[c, pl.ds(base, SLAB)])

    return deg_kernel


@functools.cache
def _msg_kernel():
    @functools.partial(
        pl.kernel,
        out_type=(
            jax.ShapeDtypeStruct((NP, HH), jnp.float32),
            jax.ShapeDtypeStruct((NP, HH), jnp.float32),
        ),
        mesh=_sc_mesh(),
        compiler_params=_SC_PARAMS,
        scratch_types=(
            pltpu.VMEM_SHARED((NP, HH), jnp.float32),
            pltpu.VMEM((IBLK, ECHUNK), jnp.int32),
            pltpu.VMEM((IBLK, ECHUNK), jnp.int32),
        )
        + tuple(pltpu.VMEM((ECHUNK, HH), jnp.float32) for _ in range(NBUF))
        + tuple(pltpu.VMEM((ECHUNK,), jnp.int32) for _ in range(2 * NBUF))
        + tuple(pltpu.SemaphoreType.DMA for _ in range(2 * NBUF)),
    )
    def msg_kernel(xs0, xs1, src_t, dst_t, acc0, acc1,
                   acc_sh, src_v, dst_v, *bufs):
        rows = bufs[:NBUF]
        idx_b = bufs[NBUF:2 * NBUF]
        src_b = bufs[2 * NBUF:3 * NBUF]
        gsem = bufs[3 * NBUF:4 * NBUF]
        ssem = bufs[4 * NBUF:5 * NBUF]
        c = lax.axis_index("c")
        s = lax.axis_index("s")
        base = s * SLAB
        rows_v0 = rows[0]

        def run(xs_hbm, acc_out):
            pltpu.sync_copy(xs_hbm.at[pl.ds(base, SLAB)],
                            acc_sh.at[pl.ds(base, SLAB)])
            plsc.subcore_barrier()

            def fill(j, p):
                for k in range(ECHUNK // 16):
                    v = src_v[j, pl.ds(k * 16, 16)]
                    src_b[p][pl.ds(k * 16, 16)] = jnp.minimum(
                        jnp.maximum(v, 0), NP - 1)
                    w = dst_v[j, pl.ds(k * 16, 16)]
                    idx_b[p][pl.ds(k * 16, 16)] = jnp.minimum(
                        jnp.maximum(w, 0), NP - 1)

            def outer(ob, _):
                pltpu.sync_copy(src_t.at[s, pl.ds(ob * IBLK, IBLK)], src_v)
                pltpu.sync_copy(dst_t.at[s, pl.ds(ob * IBLK, IBLK)], dst_v)
                g = [None] * NBUF
                sc = [None] * NBUF
                for t in range(LOOKAHEAD):
                    fill(t, t)
                    g[t] = pltpu.async_copy(
                        xs_hbm.at[src_b[t]], rows[t], gsem[t])
                for j in range(IBLK):
                    b = j % NBUF
                    g[b].wait()
                    sc[b] = pltpu.async_copy(
                        rows[b], acc_sh.at[idx_b[b]], ssem[b], add=True)
                    t = j + LOOKAHEAD
                    if t < IBLK:
                        tb = t % NBUF
                        if sc[tb] is not None:
                            sc[tb].wait()
                            sc[tb] = None
                        fill(t, tb)
                        g[tb] = pltpu.async_copy(
                            xs_hbm.at[src_b[tb]], rows[tb], gsem[tb])
                for b in range(NBUF):
                    if sc[b] is not None:
                        sc[b].wait()
                return 0

            lax.fori_loop(0, NOB, outer, 0)
            plsc.subcore_barrier()
            pltpu.sync_copy(acc_sh.at[pl.ds(base, SLAB)],
                            acc_out.at[pl.ds(base, SLAB)])

        @pl.when(c == 0)
        def _():
            run(xs0, acc0)

        @pl.when(c == 1)
        def _():
            run(xs1, acc1)

    return msg_kernel


def _first_body(x_ref, w_ref, deg_ref, xs0_ref, xs1_ref, dinv_ref):
    deg = deg_ref[0, :, 0] + deg_ref[1, :, 0] + 1.0
    dinv = lax.rsqrt(deg)[:, None]
    h = jnp.dot(x_ref[...], w_ref[...], preferred_element_type=jnp.float32)
    xs = h * dinv
    xs0_ref[...] = xs[:, :HH]
    xs1_ref[...] = xs[:, HH:]
    dinv_ref[...] = jnp.broadcast_to(dinv, (BR, HF))


@functools.cache
def _first_call():
    return pl.pallas_call(
        _first_body,
        grid=(GRID,),
        in_specs=[
            pl.BlockSpec((BR, DIN), lambda i: (i, 0)),
            pl.BlockSpec((DIN, HF), lambda i: (0, 0)),
            pl.BlockSpec((2, BR, 16), lambda i: (0, i, 0)),
        ],
        out_specs=[
            pl.BlockSpec((BR, HH), lambda i: (i, 0)),
            pl.BlockSpec((BR, HH), lambda i: (i, 0)),
            pl.BlockSpec((BR, HF), lambda i: (i, 0)),
        ],
        out_shape=[
            jax.ShapeDtypeStruct((NP, HH), jnp.float32),
            jax.ShapeDtypeStruct((NP, HH), jnp.float32),
            jax.ShapeDtypeStruct((NP, HF), jnp.float32),
        ],
    )


def _norm_relu(acc0, acc1, dinv, b, g, be):
    h = jnp.concatenate([acc0, acc1], axis=1) * dinv + b
    mu = jnp.mean(h, axis=1, keepdims=True)
    var = jnp.mean((h - mu) * (h - mu), axis=1, keepdims=True)
    y = (h - mu) * lax.rsqrt(var + 1e-5) * g + be
    return jnp.maximum(y, 0.0)


def _mid_body(acc0_ref, acc1_ref, dinv_ref, b_ref, g_ref, be_ref, w_ref,
              xs0_ref, xs1_ref):
    dinv = dinv_ref[...]
    r = _norm_relu(acc0_ref[...], acc1_ref[...], dinv,
                   b_ref[...], g_ref[...], be_ref[...])
    o = jnp.dot(r, w_ref[...], preferred_element_type=jnp.float32) * dinv
    xs0_ref[...] = o[:, :HH]
    xs1_ref[...] = o[:, HH:]


@functools.cache
def _mid_call():
    vec = pl.BlockSpec((1, HF), lambda i: (0, 0))
    return pl.pallas_call(
        _mid_body,
        grid=(GRID,),
        in_specs=[
            pl.BlockSpec((BR, HH), lambda i: (i, 0)),
            pl.BlockSpec((BR, HH), lambda i: (i, 0)),
            pl.BlockSpec((BR, HF), lambda i: (i, 0)),
            vec, vec, vec,
            pl.BlockSpec((HF, HF), lambda i: (0, 0)),
        ],
        out_specs=[
            pl.BlockSpec((BR, HH), lambda i: (i, 0)),
            pl.BlockSpec((BR, HH), lambda i: (i, 0)),
        ],
        out_shape=[
            jax.ShapeDtypeStruct((NP, HH), jnp.float32),
            jax.ShapeDtypeStruct((NP, HH), jnp.float32),
        ],
    )


def _last_body(acc0_ref, acc1_ref, dinv_ref, b_ref, g_ref, be_ref,
               wc1_ref, bc1_ref, wc2_ref, bc2_ref, out_ref):
    r = _norm_relu(acc0_ref[...], acc1_ref[...], dinv_ref[...],
                   b_ref[...], g_ref[...], be_ref[...])
    m = jnp.maximum(
        jnp.dot(r, wc1_ref[...], preferred_element_type=jnp.float32)
        + bc1_ref[...], 0.0)
    out_ref[...] = (
        jnp.dot(m, wc2_ref[...], preferred_element_type=jnp.float32)
        + bc2_ref[...])


@functools.cache
def _last_call():
    vec = pl.BlockSpec((1, HF), lambda i: (0, 0))
    return pl.pallas_call(
        _last_body,
        grid=(GRID,),
        in_specs=[
            pl.BlockSpec((BR, HH), lambda i: (i, 0)),
            pl.BlockSpec((BR, HH), lambda i: (i, 0)),
            pl.BlockSpec((BR, HF), lambda i: (i, 0)),
            vec, vec, vec,
            pl.BlockSpec((HF, HF), lambda i: (0, 0)),
            vec,
            pl.BlockSpec((HF, 1), lambda i: (0, 0)),
            pl.BlockSpec((1, 1), lambda i: (0, 0)),
        ],
        out_specs=[pl.BlockSpec((BR, 1), lambda i: (i, 0))],
        out_shape=[jax.ShapeDtypeStruct((NP, 1), jnp.float32)],
    )


def _pad_edges(idx, nway, rows, chunk, fill):
    per = EE // nway
    a = idx.reshape(nway, per)
    pad = rows * chunk - per
    a = jnp.pad(a, ((0, 0), (0, pad)), constant_values=fill)
    return a.reshape(nway, rows, chunk)


def kernel(x, edge_index, W1, b1, g1, be1, W2, b2, g2, be2, W3, b3, g3, be3,
           Wc1, bc1, Wc2, bc2):
    src = edge_index[0]
    dst = edge_index[1]
    src16 = _pad_edges(src, 16, EROWS, ECHUNK, 0)
    dst16 = _pad_edges(dst, 16, EROWS, ECHUNK, TRASH)
    dst32 = _pad_edges(dst, 32, DROWS, DCHUNK, TRASH)
    xpad = jnp.pad(x, ((0, NP - NN), (0, 0)))

    deg = _deg_kernel()(dst32)
    xs0, xs1, dinv = _first_call()(xpad, W1, deg)

    msg = _msg_kernel()
    mid = _mid_call()
    acc0, acc1 = msg(xs0, xs1, src16, dst16)
    xs0, xs1 = mid(acc0, acc1, dinv, b1.reshape(1, HF), g1.reshape(1, HF),
                   be1.reshape(1, HF), W2)
    acc0, acc1 = msg(xs0, xs1, src16, dst16)
    xs0, xs1 = mid(acc0, acc1, dinv, b2.reshape(1, HF), g2.reshape(1, HF),
                   be2.reshape(1, HF), W3)
    acc0, acc1 = msg(xs0, xs1, src16, dst16)
    (logits,) = _last_call()(acc0, acc1, dinv, b3.reshape(1, HF),
                             g3.reshape(1, HF), be3.reshape(1, HF),
                             Wc1, bc1.reshape(1, HF), Wc2,
                             bc2.reshape(1, 1))
    return logits[:NN]
